# Initial kernel scaffold; baseline (speedup 1.0000x reference)
import jax
import jax.numpy as jnp
from jax.experimental import pallas as pl


def kernel(target_word, context_word, negative_samples, word_embeddings, context_embeddings):
    raise NotImplementedError("write your pallas kernel here")


# speedup vs baseline: 4.7438x; 4.7438x over previous
import functools

import jax
import jax.numpy as jnp
from jax import lax
from jax.experimental import pallas as pl
from jax.experimental.pallas import tpu as pltpu
from jax.experimental.pallas import tpu_sc as plsc

VOCAB = 1000000
DIM = 64
BATCH = 16384
NEG = 20

NC = 2
NS = 16
L = 16
NW = NC * NS
BPW = BATCH // NW
CB = 32
NCHUNK = BPW // CB
NROWS = CB * NEG
GSUB = 128


def _sc_body(tidx_h, cidx_h, nidx_h, wtab_h, ctab_h, pos_h, negt_h,
             tidx_v, cidx_v, nidx_v, w_rows, c_rows, n_rows,
             pos_buf, neg_flat, scr, sem):
    c = lax.axis_index("c")
    s = lax.axis_index("s")
    wid = s * NC + c
    base = wid * BPW

    pltpu.sync_copy(tidx_h.at[pl.ds(base, BPW)], tidx_v)
    pltpu.sync_copy(cidx_h.at[pl.ds(base, BPW)], cidx_v)
    pltpu.sync_copy(nidx_h.at[pl.ds(base * NEG, BPW * NEG)], nidx_v)

    iota = lax.iota(jnp.int32, L)

    def chunk_body(j, carry):
        jb = j * CB
        cps = [
            pltpu.async_copy(wtab_h.at[tidx_v.at[pl.ds(jb, CB)]], w_rows, sem),
            pltpu.async_copy(ctab_h.at[cidx_v.at[pl.ds(jb, CB)]], c_rows, sem),
        ]
        for g in range(NROWS // GSUB):
            cps.append(pltpu.async_copy(
                ctab_h.at[nidx_v.at[pl.ds(j * NROWS + g * GSUB, GSUB)]],
                n_rows.at[pl.ds(g * GSUB, GSUB), :], sem))
        for cp in cps:
            cp.wait()

        def elem_body(i, carry2):
            jbi = jb + i
            wv = [w_rows[i, pl.ds(c * L, L)] for c in range(DIM // L)]
            cv = [c_rows[i, pl.ds(c * L, L)] for c in range(DIM // L)]
            p = (wv[0] * cv[0] + wv[1] * cv[1]) + (wv[2] * cv[2] + wv[3] * cv[3])
            scr[pl.ds(NEG * L, L)] = plsc.cumsum(p)
            nrow = i * NEG
            for k in range(NEG):
                nv = [n_rows[nrow + k, pl.ds(c * L, L)] for c in range(DIM // L)]
                q = (wv[0] * nv[0] + wv[1] * nv[1]) + (wv[2] * nv[2] + wv[3] * nv[3])
                scr[pl.ds(k * L, L)] = plsc.cumsum(q)
            jbi_v = jnp.broadcast_to(jbi, (L,)).astype(jnp.int32)
            t_lo = plsc.load_gather(scr, [iota * L + (L - 1)])
            plsc.store_scatter(neg_flat, [iota * BPW + jbi_v], t_lo)
            t_hi = plsc.load_gather(scr, [(iota + L) * L + (L - 1)])
            plsc.store_scatter(neg_flat, [(iota + L) * BPW + jbi_v], t_hi,
                               mask=iota < (NEG - L))
            plsc.store_scatter(pos_buf, [jbi_v], t_hi,
                               mask=iota == (NEG - L))
            return carry2

        lax.fori_loop(0, CB, elem_body, 0)
        return carry

    lax.fori_loop(0, NCHUNK, chunk_body, 0)

    pltpu.sync_copy(pos_buf, pos_h.at[pl.ds(base, BPW)])
    for k in range(NEG):
        pltpu.sync_copy(neg_flat.at[pl.ds(k * BPW, BPW)],
                        negt_h.at[k, pl.ds(base, BPW)])


_sc_scores = functools.partial(
    pl.kernel,
    out_type=(
        jax.ShapeDtypeStruct((BATCH,), jnp.float32),
        jax.ShapeDtypeStruct((NEG, BATCH), jnp.float32),
    ),
    mesh=plsc.VectorSubcoreMesh(
        core_axis_name="c", subcore_axis_name="s", num_cores=NC,
        num_subcores=NS),
    compiler_params=pltpu.CompilerParams(
        needs_layout_passes=False, use_tc_tiling_on_sc=False),
    scratch_types=[
        pltpu.VMEM((BPW,), jnp.int32),
        pltpu.VMEM((BPW,), jnp.int32),
        pltpu.VMEM((BPW * NEG,), jnp.int32),
        pltpu.VMEM((CB, DIM), jnp.float32),
        pltpu.VMEM((CB, DIM), jnp.float32),
        pltpu.VMEM((NROWS, DIM), jnp.float32),
        pltpu.VMEM((BPW,), jnp.float32),
        pltpu.VMEM((NEG * BPW,), jnp.float32),
        pltpu.VMEM((2 * NEG * L,), jnp.float32),
        pltpu.SemaphoreType.DMA,
    ],
)(_sc_body)


def _loss_body(pos_ref, negt_ref, out_ref):
    p = jnp.clip(pos_ref[...], -10.0, 10.0)
    pos_sum = jnp.sum(jax.nn.log_sigmoid(p))
    n = jnp.clip(negt_ref[...], -10.0, 10.0)
    neg_sum = jnp.sum(jax.nn.log_sigmoid(-n))
    loss = -(pos_sum / BATCH) - (neg_sum / (BATCH * NEG))
    out_ref[...] = jnp.broadcast_to(loss, (1, 1))


_loss = pl.pallas_call(
    _loss_body,
    out_shape=jax.ShapeDtypeStruct((1, 1), jnp.float32),
)


def kernel(target_word, context_word, negative_samples, word_embeddings,
           context_embeddings):
    tidx = target_word.astype(jnp.int32)
    cidx = context_word.astype(jnp.int32)
    nidx = negative_samples.astype(jnp.int32).reshape(-1)
    pos, negt = _sc_scores(tidx, cidx, nidx, word_embeddings,
                           context_embeddings)
    out = _loss(pos.reshape(BATCH // 128, 128), negt)
    return out[0, 0]


# speedup vs baseline: 4.8869x; 1.0302x over previous
import functools

import jax
import jax.numpy as jnp
from jax import lax
from jax.experimental import pallas as pl
from jax.experimental.pallas import tpu as pltpu
from jax.experimental.pallas import tpu_sc as plsc

VOCAB = 1000000
DIM = 64
DIMP = 128
BATCH = 16384
NEG = 20

NC = 2
NS = 16
L = 16
NW = NC * NS
BPW = BATCH // NW
CB = 32
NCHUNK = BPW // CB
NROWS = CB * NEG
GSUB = 128


def _sc_body(tidx_h, cidx_h, nidx_h, wtab_h, ctab_h, pos_h, negf_h,
             tidx_v, cidx_v, nidx_v, w_rows, c_rows, n_rows,
             pos_buf, neg_flat, scr, sem):
    c = lax.axis_index("c")
    s = lax.axis_index("s")
    wid = s * NC + c
    base = wid * BPW

    pltpu.sync_copy(tidx_h.at[pl.ds(base, BPW)], tidx_v)
    pltpu.sync_copy(cidx_h.at[pl.ds(base, BPW)], cidx_v)
    pltpu.sync_copy(nidx_h.at[pl.ds(base * NEG, BPW * NEG)], nidx_v)

    iota = lax.iota(jnp.int32, L)

    def chunk_body(j, carry):
        jb = j * CB
        cps = [
            pltpu.async_copy(wtab_h.at[tidx_v.at[pl.ds(jb, CB)]], w_rows, sem),
            pltpu.async_copy(ctab_h.at[cidx_v.at[pl.ds(jb, CB)]], c_rows, sem),
        ]
        for g in range(NROWS // GSUB):
            cps.append(pltpu.async_copy(
                ctab_h.at[nidx_v.at[pl.ds(j * NROWS + g * GSUB, GSUB)]],
                n_rows.at[pl.ds(g * GSUB, GSUB), :], sem))
        for cp in cps:
            cp.wait()

        def elem_body(i, carry2):
            jbi = jb + i
            wv = [w_rows[i, pl.ds(c * L, L)] for c in range(DIM // L)]
            cv = [c_rows[i, pl.ds(c * L, L)] for c in range(DIM // L)]
            p = (wv[0] * cv[0] + wv[1] * cv[1]) + (wv[2] * cv[2] + wv[3] * cv[3])
            scr[pl.ds(NEG * L, L)] = plsc.cumsum(p)
            nrow = i * NEG
            for k in range(NEG):
                nv = [n_rows[nrow + k, pl.ds(c * L, L)] for c in range(DIM // L)]
                q = (wv[0] * nv[0] + wv[1] * nv[1]) + (wv[2] * nv[2] + wv[3] * nv[3])
                scr[pl.ds(k * L, L)] = plsc.cumsum(q)
            jbi_v = jnp.broadcast_to(jbi, (L,)).astype(jnp.int32)
            t_lo = plsc.load_gather(scr, [iota * L + (L - 1)])
            plsc.store_scatter(neg_flat, [iota * BPW + jbi_v], t_lo)
            t_hi = plsc.load_gather(scr, [(iota + L) * L + (L - 1)])
            plsc.store_scatter(neg_flat, [(iota + L) * BPW + jbi_v], t_hi,
                               mask=iota < (NEG - L))
            plsc.store_scatter(pos_buf, [jbi_v], t_hi,
                               mask=iota == (NEG - L))
            return carry2

        lax.fori_loop(0, CB, elem_body, 0)
        return carry

    lax.fori_loop(0, NCHUNK, chunk_body, 0)

    pltpu.sync_copy(pos_buf, pos_h.at[pl.ds(base, BPW)])
    for k in range(NEG):
        pltpu.sync_copy(neg_flat.at[pl.ds(k * BPW, BPW)],
                        negf_h.at[pl.ds(k * BATCH + base, BPW)])


_sc_scores = functools.partial(
    pl.kernel,
    out_type=(
        jax.ShapeDtypeStruct((BATCH,), jnp.float32),
        jax.ShapeDtypeStruct((NEG * BATCH,), jnp.float32),
    ),
    mesh=plsc.VectorSubcoreMesh(
        core_axis_name="c", subcore_axis_name="s", num_cores=NC,
        num_subcores=NS),
    compiler_params=pltpu.CompilerParams(
        needs_layout_passes=False, use_tc_tiling_on_sc=True),
    scratch_types=[
        pltpu.VMEM((BPW,), jnp.int32),
        pltpu.VMEM((BPW,), jnp.int32),
        pltpu.VMEM((BPW * NEG,), jnp.int32),
        pltpu.VMEM((CB, DIMP), jnp.float32),
        pltpu.VMEM((CB, DIMP), jnp.float32),
        pltpu.VMEM((NROWS, DIMP), jnp.float32),
        pltpu.VMEM((BPW,), jnp.float32),
        pltpu.VMEM((NEG * BPW,), jnp.float32),
        pltpu.VMEM((2 * NEG * L,), jnp.float32),
        pltpu.SemaphoreType.DMA,
    ],
)(_sc_body)


def _loss_body(pos_ref, negt_ref, out_ref):
    p = jnp.clip(pos_ref[...], -10.0, 10.0)
    pos_sum = jnp.sum(jax.nn.log_sigmoid(p))
    n = jnp.clip(negt_ref[...], -10.0, 10.0)
    neg_sum = jnp.sum(jax.nn.log_sigmoid(-n))
    loss = -(pos_sum / BATCH) - (neg_sum / (BATCH * NEG))
    out_ref[...] = jnp.broadcast_to(loss, (1, 1))


_loss = pl.pallas_call(
    _loss_body,
    out_shape=jax.ShapeDtypeStruct((1, 1), jnp.float32),
)


def kernel(target_word, context_word, negative_samples, word_embeddings,
           context_embeddings):
    tidx = target_word.astype(jnp.int32)
    cidx = context_word.astype(jnp.int32)
    nidx = negative_samples.astype(jnp.int32).reshape(-1)
    wpad = jnp.pad(word_embeddings, ((0, 0), (0, DIMP - DIM)))
    cpad = jnp.pad(context_embeddings, ((0, 0), (0, DIMP - DIM)))
    pos, negf = _sc_scores(tidx, cidx, nidx, wpad, cpad)
    out = _loss(pos.reshape(BATCH // 128, 128), negf.reshape(NEG, BATCH))
    return out[0, 0]
